# baseline (speedup 1.0000x reference)
import dataclasses
import functools

import jax
import jax.numpy as jnp
from jax.experimental import pallas as pl
from jax.experimental.pallas import tpu as pltpu
from jax.experimental.pallas import tpu_sc as plsc

N_TOKEN = 100000
D_PROJ = 1024
CUT0, CUT1, CUT2 = 20000, 60000, 100000
D0, D1, D2 = 1024, 256, 64
N_TOK_TOTAL = 8192

TOK_BLOCK = 2048

NC, NS = 2, 16
NW = NC * NS
TOK_W = N_TOK_TOTAL // NW

CH = 64
NCHUNK = TOK_W // CH
CH_SHIFT, CH_MASK = 6, CH - 1


def _sc_gather(toks, w0, w1, w2):
    mesh = plsc.VectorSubcoreMesh(core_axis_name="c", subcore_axis_name="s")

    out_type = (
        jax.ShapeDtypeStruct((N_TOK_TOTAL, D0), jnp.float32),
        jax.ShapeDtypeStruct((N_TOK_TOTAL, D1), jnp.float32),
        jax.ShapeDtypeStruct((N_TOK_TOTAL, 2 * D2), jnp.float32),
    )

    scratch_types = [
        pltpu.VMEM((TOK_W,), jnp.float32),
        pltpu.VMEM((NCHUNK, CH), jnp.int32),
        pltpu.VMEM((NCHUNK, CH), jnp.int32),
        pltpu.VMEM((NCHUNK, CH), jnp.int32),
        pltpu.VMEM((NCHUNK, CH), jnp.int32),
        pltpu.VMEM((NCHUNK, CH), jnp.int32),
        pltpu.VMEM((NCHUNK, CH), jnp.int32),
        pltpu.VMEM((CH, D0), jnp.float32),
        pltpu.VMEM((CH, D1), jnp.float32),
        pltpu.VMEM((CH, 2 * D2), jnp.float32),
    ]

    cp = pltpu.CompilerParams()
    if "needs_layout_passes" in pltpu.CompilerParams.__dataclass_fields__:
        cp = dataclasses.replace(cp, needs_layout_passes=False)

    @functools.partial(pl.kernel, out_type=out_type, mesh=mesh,
                       scratch_types=scratch_types, compiler_params=cp)
    def gather_kernel(tokh, w0h, w1h, w2h, e0h, e1h, e2h,
                      tokv, g0v, s0v, g1v, s1v, g2v, s2v,
                      buf0, buf1, buf2):
        wid = jax.lax.axis_index("s") * NC + jax.lax.axis_index("c")
        base = wid * TOK_W

        per_row = 2048 // TOK_W
        pltpu.sync_copy(
            tokh.at[wid // per_row, pl.ds((wid % per_row) * TOK_W, TOK_W)],
            tokv)

        lanes = jax.lax.iota(jnp.int32, 16)
        tables = (
            (g0v, s0v, 0, CUT0, lambda t: t, w0h, e0h, buf0),
            (g1v, s1v, CUT0, CUT1, lambda t: t - CUT0, w1h, e1h, buf1),
            (g2v, s2v, CUT1, CUT2, lambda t: t - CUT1, w2h, e2h, buf2),
        )

        cnts = [jnp.int32(0), jnp.int32(0), jnp.int32(0)]
        for v in range(TOK_W // 16):
            tok = tokv[pl.ds(v * 16, 16)].astype(jnp.int32)
            gpos = base + v * 16 + lanes
            for t, (gv, sv, lo, hi, to_row, _, _, _) in enumerate(tables):
                m = jnp.logical_and(tok >= lo, tok < hi)
                mi = m.astype(jnp.int32)
                pos = cnts[t] + jnp.cumsum(mi) - 1
                plsc.store_scatter(gv, [pos >> CH_SHIFT, pos & CH_MASK],
                                   to_row(tok), mask=m)
                plsc.store_scatter(sv, [pos >> CH_SHIFT, pos & CH_MASK],
                                   gpos, mask=m)
                cnts[t] = cnts[t] + jnp.sum(mi)

        zeros16 = lanes * 0
        for t, (gv, sv, *_rest) in enumerate(tables):
            cnt = cnts[t]
            padded = ((cnt + CH - 1) >> CH_SHIFT) << CH_SHIFT
            gfirst = plsc.load_gather(gv, [zeros16, zeros16])
            sfirst = plsc.load_gather(sv, [zeros16, zeros16])
            for k in range(CH // 16):
                p = cnt + k * 16 + lanes
                pm = p < padded
                plsc.store_scatter(gv, [p >> CH_SHIFT, p & CH_MASK],
                                   gfirst, mask=pm)
                plsc.store_scatter(sv, [p >> CH_SHIFT, p & CH_MASK],
                                   sfirst, mask=pm)

        for t, (gv, sv, _, _, _, wh, eh, buf) in enumerate(tables):
            used = (cnts[t] + CH - 1) >> CH_SHIFT
            for c in range(NCHUNK):
                @pl.when(c < used)
                def _():
                    pltpu.sync_copy(wh.at[gv.at[c]], buf)
                    pltpu.sync_copy(buf, eh.at[sv.at[c]])

    return gather_kernel(toks, w0, w1, w2)


def _tc_project_body(inp_ref, e0_ref, e1_ref, e2_ref, p0_ref, p1_ref, p2_ref,
                     out_ref):
    i = pl.program_id(0)
    per_row = 2048 // TOK_BLOCK
    tokr = inp_ref[pl.ds(i // per_row, 1), pl.ds((i % per_row) * TOK_BLOCK,
                                                 TOK_BLOCK)]
    tok = jnp.transpose(tokr)
    m0 = tok < CUT0
    m1 = tok < CUT1
    a0 = e0_ref[...].astype(jnp.bfloat16)
    a1 = e1_ref[...].astype(jnp.bfloat16)
    a2 = e2_ref[:, :D2].astype(jnp.bfloat16)
    d0 = jnp.dot(a0, p0_ref[...], preferred_element_type=jnp.float32)
    d1 = jnp.dot(a1, p1_ref[...], preferred_element_type=jnp.float32)
    d2 = jnp.dot(a2, p2_ref[...], preferred_element_type=jnp.float32)
    out_ref[...] = jnp.where(m0, d0, jnp.where(m1, d1, d2))


def _tc_project(inp, e0, e1, e2, p0b, p1b, p2b):
    grid = (N_TOK_TOTAL // TOK_BLOCK,)
    per_row = 2048 // TOK_BLOCK
    return pl.pallas_call(
        _tc_project_body,
        grid=grid,
        in_specs=[
            pl.BlockSpec((4, 2048), lambda i: (0, 0)),
            pl.BlockSpec((TOK_BLOCK, D0), lambda i: (i, 0)),
            pl.BlockSpec((TOK_BLOCK, D1), lambda i: (i, 0)),
            pl.BlockSpec((TOK_BLOCK, 2 * D2), lambda i: (i, 0)),
            pl.BlockSpec((D0, D_PROJ), lambda i: (0, 0)),
            pl.BlockSpec((D1, D_PROJ), lambda i: (0, 0)),
            pl.BlockSpec((D2, D_PROJ), lambda i: (0, 0)),
        ],
        out_specs=pl.BlockSpec((TOK_BLOCK, D_PROJ), lambda i: (i, 0)),
        out_shape=jax.ShapeDtypeStruct((N_TOK_TOTAL, D_PROJ), jnp.float32),
        compiler_params=pltpu.CompilerParams(
            dimension_semantics=("parallel",)),
    )(inp, e0, e1, e2, p0b, p1b, p2b)


def kernel(inp, w0, w1, w2, p0, p1, p2):
    w2p = jnp.pad(w2, ((0, 0), (0, 2 * D2 - D2)))
    inp8 = jnp.pad(inp, ((0, 4), (0, 0))).astype(jnp.float32)
    e0, e1, e2 = _sc_gather(inp8, w0, w1, w2p)

    scale = D_PROJ ** 0.5
    out = _tc_project(inp, e0, e1, e2,
                      (p0 * scale).astype(jnp.bfloat16),
                      (p1 * scale).astype(jnp.bfloat16),
                      (p2 * scale).astype(jnp.bfloat16))
    return out.reshape(inp.shape + (D_PROJ,))


# speedup vs baseline: 1.0211x; 1.0211x over previous
import dataclasses
import functools

import jax
import jax.numpy as jnp
from jax.experimental import pallas as pl
from jax.experimental.pallas import tpu as pltpu
from jax.experimental.pallas import tpu_sc as plsc

N_TOKEN = 100000
D_PROJ = 1024
CUT0, CUT1, CUT2 = 20000, 60000, 100000
D0, D1, D2 = 1024, 256, 64
N_TOK_TOTAL = 8192

TOK_BLOCK = 1024

NC, NS = 2, 16
NW = NC * NS
TOK_W = N_TOK_TOTAL // NW

CH0, NCHUNK0, SH0 = 64, TOK_W // 64, 6
CH1, NCHUNK1, SH1 = 64, TOK_W // 64, 6
CH2, NCHUNK2, SH2 = 64, TOK_W // 64, 6


def _sc_gather(toks, w0, w1, w2):
    mesh = plsc.VectorSubcoreMesh(core_axis_name="c", subcore_axis_name="s")

    out_type = (
        jax.ShapeDtypeStruct((N_TOK_TOTAL, D0), jnp.float32),
        jax.ShapeDtypeStruct((N_TOK_TOTAL, D1), jnp.float32),
        jax.ShapeDtypeStruct((N_TOK_TOTAL, 2 * D2), jnp.float32),
    )

    scratch_types = [
        pltpu.VMEM((TOK_W,), jnp.int32),
        pltpu.VMEM((NCHUNK0, CH0), jnp.int32),
        pltpu.VMEM((NCHUNK0, CH0), jnp.int32),
        pltpu.VMEM((NCHUNK1, CH1), jnp.int32),
        pltpu.VMEM((NCHUNK1, CH1), jnp.int32),
        pltpu.VMEM((NCHUNK2, CH2), jnp.int32),
        pltpu.VMEM((NCHUNK2, CH2), jnp.int32),
        pltpu.VMEM((CH0, D0), jnp.float32),
        pltpu.VMEM((CH1, D1), jnp.float32),
        pltpu.VMEM((CH2, 2 * D2), jnp.float32),
    ]

    cp = pltpu.CompilerParams()
    if "needs_layout_passes" in pltpu.CompilerParams.__dataclass_fields__:
        cp = dataclasses.replace(cp, needs_layout_passes=False)

    @functools.partial(pl.kernel, out_type=out_type, mesh=mesh,
                       scratch_types=scratch_types, compiler_params=cp)
    def gather_kernel(tokh, w0h, w1h, w2h, e0h, e1h, e2h,
                      tokv, g0v, s0v, g1v, s1v, g2v, s2v,
                      buf0, buf1, buf2):
        wid = jax.lax.axis_index("s") * NC + jax.lax.axis_index("c")
        base = wid * TOK_W

        per_row = 2048 // TOK_W
        pltpu.sync_copy(
            tokh.at[wid // per_row, pl.ds((wid % per_row) * TOK_W, TOK_W)],
            tokv)

        lanes = jax.lax.iota(jnp.int32, 16)
        tables = (
            (g0v, s0v, 0, CUT0, lambda t: t, w0h, e0h, buf0,
             CH0, NCHUNK0, SH0),
            (g1v, s1v, CUT0, CUT1, lambda t: t - CUT0, w1h, e1h, buf1,
             CH1, NCHUNK1, SH1),
            (g2v, s2v, CUT1, CUT2, lambda t: t - CUT1, w2h, e2h, buf2,
             CH2, NCHUNK2, SH2),
        )

        cnts = [jnp.int32(0), jnp.int32(0), jnp.int32(0)]
        for v in range(TOK_W // 16):
            tok = tokv[pl.ds(v * 16, 16)]
            gpos = base + v * 16 + lanes
            for t, (gv, sv, lo, hi, to_row, _, _, _, ch, _, sh) in (
                    enumerate(tables)):
                m = jnp.logical_and(tok >= lo, tok < hi)
                mi = m.astype(jnp.int32)
                pos = cnts[t] + jnp.cumsum(mi) - 1
                plsc.store_scatter(gv, [pos >> sh, pos & (ch - 1)],
                                   to_row(tok), mask=m)
                plsc.store_scatter(sv, [pos >> sh, pos & (ch - 1)],
                                   gpos, mask=m)
                cnts[t] = cnts[t] + jnp.sum(mi)

        zeros16 = lanes * 0
        for t, (gv, sv, _, _, _, _, _, _, ch, _, sh) in enumerate(tables):
            cnt = cnts[t]
            padded = ((cnt + ch - 1) >> sh) << sh
            gfirst = plsc.load_gather(gv, [zeros16, zeros16])
            sfirst = plsc.load_gather(sv, [zeros16, zeros16])
            for k in range(ch // 16):
                p = cnt + k * 16 + lanes
                pm = p < padded
                plsc.store_scatter(gv, [p >> sh, p & (ch - 1)],
                                   gfirst, mask=pm)
                plsc.store_scatter(sv, [p >> sh, p & (ch - 1)],
                                   sfirst, mask=pm)

        for t, (gv, sv, _, _, _, wh, eh, buf, ch, nchunk, sh) in (
                enumerate(tables)):
            used = (cnts[t] + ch - 1) >> sh
            for c in range(nchunk):
                @pl.when(c < used)
                def _(c=c):
                    pltpu.sync_copy(wh.at[gv.at[c]], buf)
                    pltpu.sync_copy(buf, eh.at[sv.at[c]])

    return gather_kernel(toks, w0, w1, w2)


def _tc_project_body(inp_ref, e0_ref, e1_ref, e2_ref, p0_ref, p1_ref, p2_ref,
                     out_ref):
    i = pl.program_id(0)
    per_row = 2048 // TOK_BLOCK
    tokr = inp_ref[pl.ds(i // per_row, 1), pl.ds((i % per_row) * TOK_BLOCK,
                                                 TOK_BLOCK)]
    tok = jnp.transpose(tokr)
    m0 = tok < CUT0
    m1 = tok < CUT1
    a0 = e0_ref[...].astype(jnp.bfloat16)
    a1 = e1_ref[...].astype(jnp.bfloat16)
    a2 = e2_ref[:, :D2].astype(jnp.bfloat16)
    d0 = jnp.dot(a0, p0_ref[...], preferred_element_type=jnp.float32)
    d1 = jnp.dot(a1, p1_ref[...], preferred_element_type=jnp.float32)
    d2 = jnp.dot(a2, p2_ref[...], preferred_element_type=jnp.float32)
    out_ref[...] = jnp.where(m0, d0, jnp.where(m1, d1, d2))


def _tc_project(inp, e0, e1, e2, p0b, p1b, p2b):
    grid = (N_TOK_TOTAL // TOK_BLOCK,)
    per_row = 2048 // TOK_BLOCK
    return pl.pallas_call(
        _tc_project_body,
        grid=grid,
        in_specs=[
            pl.BlockSpec((4, 2048), lambda i: (0, 0)),
            pl.BlockSpec((TOK_BLOCK, D0), lambda i: (i, 0)),
            pl.BlockSpec((TOK_BLOCK, D1), lambda i: (i, 0)),
            pl.BlockSpec((TOK_BLOCK, 2 * D2), lambda i: (i, 0)),
            pl.BlockSpec((D0, D_PROJ), lambda i: (0, 0)),
            pl.BlockSpec((D1, D_PROJ), lambda i: (0, 0)),
            pl.BlockSpec((D2, D_PROJ), lambda i: (0, 0)),
        ],
        out_specs=pl.BlockSpec((TOK_BLOCK, D_PROJ), lambda i: (i, 0)),
        out_shape=jax.ShapeDtypeStruct((N_TOK_TOTAL, D_PROJ), jnp.float32),
        compiler_params=pltpu.CompilerParams(
            dimension_semantics=("parallel",)),
    )(inp, e0, e1, e2, p0b, p1b, p2b)


def kernel(inp, w0, w1, w2, p0, p1, p2):
    w2p = jnp.pad(w2, ((0, 0), (0, 2 * D2 - D2)))
    inp8 = jnp.pad(inp, ((0, 4), (0, 0)))
    e0, e1, e2 = _sc_gather(inp8, w0, w1, w2p)

    scale = D_PROJ ** 0.5
    out = _tc_project(inp, e0, e1, e2,
                      (p0 * scale).astype(jnp.bfloat16),
                      (p1 * scale).astype(jnp.bfloat16),
                      (p2 * scale).astype(jnp.bfloat16))
    return out.reshape(inp.shape + (D_PROJ,))
